# baseline (speedup 1.0000x reference)
import functools

import jax
import jax.numpy as jnp
from jax import lax
from jax.experimental import pallas as pl
from jax.experimental.pallas import tpu as pltpu
from jax.experimental.pallas import tpu_sc as plsc

_B, _S, _V = 16, 128, 5000
_N = 4
_KEY_HI, _KEY_LO = 0, 42

_L = 16
_RPW = 64
_RC = 8
_NCHUNK = _RPW // _RC
_NVEC = _V // _L
_TAIL_BASE = _V - _L
_TAIL_DUP = _NVEC * _L - _TAIL_BASE


def _threefry2x32(x0, x1):
    k0 = jnp.uint32(_KEY_HI)
    k1 = jnp.uint32(_KEY_LO)
    ks2 = jnp.uint32(0x1BD11BDA) ^ k0 ^ k1
    ks = (k0, k1, ks2)
    rots = ((13, 15, 26, 6), (17, 29, 16, 24))
    x0 = x0 + k0
    x1 = x1 + k1
    for i in range(5):
        for r in rots[i % 2]:
            x0 = x0 + x1
            x1 = (x1 << r) | (x1 >> (32 - r))
            x1 = x1 ^ x0
        x0 = x0 + ks[(i + 1) % 3]
        x1 = x1 + ks[(i + 2) % 3] + jnp.uint32(i + 1)
    return x0, x1


def _splat_f(x):
    return jnp.broadcast_to(jnp.float32(x), (_L,))


def _splat_i(x):
    return jnp.broadcast_to(jnp.int32(x), (_L,))


def _sc_top2_body(logit_hbm, m1_hbm, i1_hbm, m2_hbm, i2_hbm,
                  buf, om1, oi1, om2, oi2, tmpf, tmpi, sem):
    cid = lax.axis_index("c")
    sid = lax.axis_index("s")
    wid = sid * 2 + cid
    b = wid // 2
    s0 = (wid % 2) * _RPW

    lane = lax.iota(jnp.int32, _L)
    big = _splat_i(_V)
    neg = _splat_f(-jnp.inf)

    def start_chunk(chunk, slot):
        return pltpu.make_async_copy(
            logit_hbm.at[b, pl.ds(s0 + chunk * _RC, _RC)],
            buf.at[slot], sem.at[slot])

    start_chunk(0, 0).start()

    acc = [neg, _splat_i(0), neg, _splat_i(0)]

    for chunk in range(_NCHUNK):
        slot = chunk % 2
        if chunk + 1 < _NCHUNK:
            start_chunk(chunk + 1, (chunk + 1) % 2).start()
        start_chunk(chunk, slot).wait()

        for k in range(_RC):
            row_local = chunk * _RC + k

            def step(j, carry, _slot=slot, _k=k):
                a1, p1, a2, p2 = carry
                c = buf[_slot, _k, pl.ds(j * _L, _L)]
                base = jnp.broadcast_to(j * _L, (_L,))
                gt1 = c > a1
                gt2 = c > a2
                a2n = jnp.where(gt1, a1, jnp.where(gt2, c, a2))
                p2n = jnp.where(gt1, p1, jnp.where(gt2, base, p2))
                a1n = jnp.where(gt1, c, a1)
                p1n = jnp.where(gt1, base, p1)
                return a1n, p1n, a2n, p2n

            init = (neg, big, neg, big)
            a1, p1, a2, p2 = lax.fori_loop(0, _NVEC, step, init)

            c = buf[slot, k, pl.ds(_TAIL_BASE, _L)]
            c = jnp.where(lane < _TAIL_DUP, neg, c)
            base = _splat_i(_TAIL_BASE)
            gt1 = c > a1
            gt2 = c > a2
            a2 = jnp.where(gt1, a1, jnp.where(gt2, c, a2))
            p2 = jnp.where(gt1, p1, jnp.where(gt2, base, p2))
            a1 = jnp.where(gt1, c, a1)
            p1 = jnp.where(gt1, base, p1)

            i1l = p1 + lane
            i2l = p2 + lane

            last = _splat_i(_L - 1)

            def _splat_max_f(v):
                tmpf[...] = plsc.cummax(v)
                return plsc.load_gather(tmpf, [last])

            def _splat_min_i(v):
                tmpi[...] = plsc.cummax(-v)
                return -plsc.load_gather(tmpi, [last])

            m1 = _splat_max_f(a1)
            i1 = _splat_min_i(jnp.where(a1 == m1, i1l, big))
            win = jnp.logical_and(a1 == m1, i1l == i1)
            c2 = jnp.where(win, a2, a1)
            c2i = jnp.where(win, i2l, i1l)
            m2 = _splat_max_f(c2)
            i2 = _splat_min_i(jnp.where(c2 == m2, c2i, big))

            dep = lane == _splat_i(row_local % _L)
            acc[0] = jnp.where(dep, m1, acc[0])
            acc[1] = jnp.where(dep, i1, acc[1])
            acc[2] = jnp.where(dep, m2, acc[2])
            acc[3] = jnp.where(dep, i2, acc[3])

            if row_local % _L == _L - 1:
                g = (row_local // _L) * _L
                om1[pl.ds(g, _L)] = acc[0]
                oi1[pl.ds(g, _L)] = acc[1]
                om2[pl.ds(g, _L)] = acc[2]
                oi2[pl.ds(g, _L)] = acc[3]
                acc = [neg, _splat_i(0), neg, _splat_i(0)]

    base_out = wid * _RPW
    pltpu.sync_copy(om1, m1_hbm.at[pl.ds(base_out, _RPW)])
    pltpu.sync_copy(oi1, i1_hbm.at[pl.ds(base_out, _RPW)])
    pltpu.sync_copy(om2, m2_hbm.at[pl.ds(base_out, _RPW)])
    pltpu.sync_copy(oi2, i2_hbm.at[pl.ds(base_out, _RPW)])


def _finish_body(lens_ref, m1_ref, i1_ref, m2_ref, i2_ref, tgt_ref, out_ref):
    m1 = m1_ref[...]
    m2 = m2_ref[...]
    i1 = i1_ref[...]
    i2 = i2_ref[...]
    tgt = tgt_ref[...]
    lens = lens_ref[...]

    b_iota = lax.broadcasted_iota(jnp.int32, (_B, _S), 0)
    s_iota = lax.broadcasted_iota(jnp.int32, (_B, _S), 1)
    pad = s_iota >= lens

    a_list = []
    err_list = []
    for n in range(_N):
        flat = ((n * _B + b_iota) * _S + s_iota) * _V + i1
        o0, o1 = _threefry2x32(jnp.zeros((_B, _S), jnp.uint32),
                               flat.astype(jnp.uint32))
        bits = o0 ^ o1
        masked = (bits >> 31) == 0

        sel_v = jnp.where(masked, m2, m1)
        sel_v = jnp.where(pad, 0.0, sel_v)
        a_list.append(jnp.sum(sel_v, axis=-1, keepdims=True))

        pred = jnp.where(masked, i2, i1)
        err_list.append(jnp.sum(
            jnp.where(pad, 0.0, (tgt != pred).astype(jnp.float32)),
            axis=-1, keepdims=True))

    a = jnp.concatenate(a_list, axis=1)
    err = jnp.concatenate(err_list, axis=1)

    md = jnp.max(a, axis=1, keepdims=True)
    w = jnp.exp(a - md)
    normal = w / jnp.sum(w, axis=1, keepdims=True)
    dev = err - jnp.mean(err, axis=1, keepdims=True)
    loss_b = jnp.sum(normal * dev, axis=1, keepdims=True)
    out_ref[...] = jnp.sum(loss_b, axis=0, keepdims=True) / _B


def kernel(logit, tgt, tgt_lens):
    mesh = plsc.VectorSubcoreMesh(core_axis_name="c", subcore_axis_name="s")
    nrows = _B * _S

    sc_top2 = pl.kernel(
        _sc_top2_body,
        mesh=mesh,
        out_type=[
            jax.ShapeDtypeStruct((nrows,), jnp.float32),
            jax.ShapeDtypeStruct((nrows,), jnp.int32),
            jax.ShapeDtypeStruct((nrows,), jnp.float32),
            jax.ShapeDtypeStruct((nrows,), jnp.int32),
        ],
        scratch_types=[
            pltpu.VMEM((2, _RC, _V), jnp.float32),
            pltpu.VMEM((_RPW,), jnp.float32),
            pltpu.VMEM((_RPW,), jnp.int32),
            pltpu.VMEM((_RPW,), jnp.float32),
            pltpu.VMEM((_RPW,), jnp.int32),
            pltpu.VMEM((_L,), jnp.float32),
            pltpu.VMEM((_L,), jnp.int32),
            pltpu.SemaphoreType.DMA((2,)),
        ],
        compiler_params=pltpu.CompilerParams(needs_layout_passes=False),
    )

    m1, i1, m2, i2 = sc_top2(logit)
    m1 = m1.reshape(_B, _S)
    i1 = i1.reshape(_B, _S)
    m2 = m2.reshape(_B, _S)
    i2 = i2.reshape(_B, _S)

    loss = pl.pallas_call(
        _finish_body,
        in_specs=[
            pl.BlockSpec((_B, 1), lambda: (0, 0)),
            pl.BlockSpec((_B, _S), lambda: (0, 0)),
            pl.BlockSpec((_B, _S), lambda: (0, 0)),
            pl.BlockSpec((_B, _S), lambda: (0, 0)),
            pl.BlockSpec((_B, _S), lambda: (0, 0)),
            pl.BlockSpec((_B, _S), lambda: (0, 0)),
        ],
        out_specs=pl.BlockSpec((1, 1), lambda: (0, 0)),
        out_shape=jax.ShapeDtypeStruct((1, 1), jnp.float32),
    )(tgt_lens.reshape(_B, 1), m1, i1, m2, i2, tgt)
    return loss[0, 0]


# speedup vs baseline: 4.9244x; 4.9244x over previous
import jax
import jax.numpy as jnp
from jax import lax
from jax.experimental import pallas as pl
from jax.experimental.pallas import tpu as pltpu

_B, _S, _V = 16, 128, 5000
_N = 4
_NC = _V // 8
_KEY_HI, _KEY_LO = 0, 42


def _threefry2x32(x0, x1):
    k0 = jnp.uint32(_KEY_HI)
    k1 = jnp.uint32(_KEY_LO)
    ks2 = jnp.uint32(0x1BD11BDA) ^ k0 ^ k1
    ks = (k0, k1, ks2)
    rots = ((13, 15, 26, 6), (17, 29, 16, 24))
    x0 = x0 + k0
    x1 = x1 + k1
    for i in range(5):
        for r in rots[i % 2]:
            x0 = x0 + x1
            x1 = (x1 << r) | (x1 >> (32 - r))
            x1 = x1 ^ x0
        x0 = x0 + ks[(i + 1) % 3]
        x1 = x1 + ks[(i + 2) % 3] + jnp.uint32(i + 1)
    return x0, x1


def _mwer_body(len_ref, xt_ref, tgt_ref, out_ref):
    b = pl.program_id(0)
    neg = jnp.full((8, _S), -jnp.inf, jnp.float32)
    zero = jnp.zeros((8, _S), jnp.int32)

    def step(j, carry):
        a1, p1, a2, p2 = carry
        c = xt_ref[0, pl.ds(j * 8, 8), :]
        bj = jnp.full((8, _S), j, jnp.int32)
        gt1 = c > a1
        gt2 = c > a2
        a2n = jnp.where(gt1, a1, jnp.where(gt2, c, a2))
        p2n = jnp.where(gt1, p1, jnp.where(gt2, bj, p2))
        a1n = jnp.where(gt1, c, a1)
        p1n = jnp.where(gt1, bj, p1)
        return a1n, p1n, a2n, p2n

    a1, p1, a2, p2 = lax.fori_loop(
        0, _NC, step, (neg, zero, neg, zero), unroll=8)

    row8 = lax.broadcasted_iota(jnp.int32, (8, _S), 0)
    i1s = p1 * 8 + row8
    i2s = p2 * 8 + row8

    big = _V
    m1 = jnp.max(a1, axis=0, keepdims=True)
    i1 = jnp.min(jnp.where(a1 == m1, i1s, big), axis=0, keepdims=True)
    win = jnp.logical_and(a1 == m1, i1s == i1)
    c2 = jnp.where(win, a2, a1)
    c2i = jnp.where(win, i2s, i1s)
    m2 = jnp.max(c2, axis=0, keepdims=True)
    i2 = jnp.min(jnp.where(c2 == m2, c2i, big), axis=0, keepdims=True)

    n_iota = lax.broadcasted_iota(jnp.int32, (_N, _S), 0)
    s_iota = lax.broadcasted_iota(jnp.int32, (_N, _S), 1)
    i1r = jnp.broadcast_to(i1, (_N, _S))
    flat = ((n_iota * _B + b) * _S + s_iota) * _V + i1r
    o0, o1 = _threefry2x32(jnp.zeros((_N, _S), jnp.uint32),
                           flat.astype(jnp.uint32))
    bits = o0 ^ o1
    masked = (bits >> 31) == 0

    pad = s_iota >= len_ref[b]
    v1 = jnp.broadcast_to(m1, (_N, _S))
    v2 = jnp.broadcast_to(m2, (_N, _S))
    i2r = jnp.broadcast_to(i2, (_N, _S))

    sel_v = jnp.where(masked, v2, v1)
    sel_v = jnp.where(pad, 0.0, sel_v)
    a = jnp.sum(sel_v, axis=-1, keepdims=True)

    pred = jnp.where(masked, i2r, i1r)
    tgt = jnp.broadcast_to(tgt_ref[0, 0].reshape(1, _S), (_N, _S))
    err = jnp.sum(
        jnp.where(pad, 0.0, (tgt != pred).astype(jnp.float32)),
        axis=-1, keepdims=True)

    md = jnp.max(a, axis=0, keepdims=True)
    w = jnp.exp(a - md)
    normal = w / jnp.sum(w, axis=0, keepdims=True)
    dev = err - jnp.mean(err, axis=0, keepdims=True)
    out_ref[0] = jnp.sum(normal * dev, axis=0, keepdims=True)


def kernel(logit, tgt, tgt_lens):
    xt = jnp.transpose(logit, (0, 2, 1))
    tgt3 = tgt.reshape(_B, 1, _S)
    loss = pl.pallas_call(
        _mwer_body,
        grid=(_B,),
        in_specs=[
            pl.BlockSpec(memory_space=pltpu.SMEM),
            pl.BlockSpec((1, _V, _S), lambda b: (b, 0, 0)),
            pl.BlockSpec((1, 1, _S), lambda b: (b, 0, 0)),
        ],
        out_specs=pl.BlockSpec((1, 1, 1), lambda b: (b, 0, 0)),
        out_shape=jax.ShapeDtypeStruct((_B, 1, 1), jnp.float32),
    )(tgt_lens, xt, tgt3)
    return jnp.mean(loss)


# speedup vs baseline: 5.9087x; 1.1999x over previous
import jax
import jax.numpy as jnp
from jax import lax
from jax.experimental import pallas as pl
from jax.experimental.pallas import tpu as pltpu

_B, _S, _V = 16, 128, 5000
_N = 4
_NC = _V // 8
_NBANK = 5
_KEY_HI, _KEY_LO = 0, 42


def _threefry2x32(x0, x1):
    k0 = jnp.uint32(_KEY_HI)
    k1 = jnp.uint32(_KEY_LO)
    ks2 = jnp.uint32(0x1BD11BDA) ^ k0 ^ k1
    ks = (k0, k1, ks2)
    rots = ((13, 15, 26, 6), (17, 29, 16, 24))
    x0 = x0 + k0
    x1 = x1 + k1
    for i in range(5):
        for r in rots[i % 2]:
            x0 = x0 + x1
            x1 = (x1 << r) | (x1 >> (32 - r))
            x1 = x1 ^ x0
        x0 = x0 + ks[(i + 1) % 3]
        x1 = x1 + ks[(i + 2) % 3] + jnp.uint32(i + 1)
    return x0, x1


def _mwer_body(len_ref, xt_ref, tgt_ref, out_ref):
    b = pl.program_id(0)
    neg = jnp.full((8, _S), -jnp.inf, jnp.float32)
    zero = jnp.zeros((8, _S), jnp.int32)

    nb = _NBANK
    per = _NC // nb

    def step(j, carry):
        new = []
        for k in range(nb):
            a1, p1, a2, p2 = carry[4 * k:4 * k + 4]
            cj = k * per + j
            c = xt_ref[0, pl.ds(cj * 8, 8), :]
            bj = jnp.full((8, _S), cj, jnp.int32)
            gt1 = c > a1
            gt2 = c > a2
            new.append(jnp.where(gt1, c, a1))
            new.append(jnp.where(gt1, bj, p1))
            new.append(jnp.where(gt1, a1, jnp.where(gt2, c, a2)))
            new.append(jnp.where(gt1, p1, jnp.where(gt2, bj, p2)))
        return tuple(new)

    init = (neg, zero, neg, zero) * nb
    res = lax.fori_loop(0, per, step, init, unroll=4)

    row8 = lax.broadcasted_iota(jnp.int32, (8, _S), 0)
    a1 = jnp.concatenate([res[4 * k] for k in range(nb)], axis=0)
    i1s = jnp.concatenate(
        [res[4 * k + 1] * 8 + row8 for k in range(nb)], axis=0)
    a2 = jnp.concatenate([res[4 * k + 2] for k in range(nb)], axis=0)
    i2s = jnp.concatenate(
        [res[4 * k + 3] * 8 + row8 for k in range(nb)], axis=0)

    big = _V
    m1 = jnp.max(a1, axis=0, keepdims=True)
    i1 = jnp.min(jnp.where(a1 == m1, i1s, big), axis=0, keepdims=True)
    win = jnp.logical_and(a1 == m1, i1s == i1)
    c2 = jnp.where(win, a2, a1)
    c2i = jnp.where(win, i2s, i1s)
    m2 = jnp.max(c2, axis=0, keepdims=True)
    i2 = jnp.min(jnp.where(c2 == m2, c2i, big), axis=0, keepdims=True)

    n_iota = lax.broadcasted_iota(jnp.int32, (_N, _S), 0)
    s_iota = lax.broadcasted_iota(jnp.int32, (_N, _S), 1)
    i1r = jnp.broadcast_to(i1, (_N, _S))
    flat = ((n_iota * _B + b) * _S + s_iota) * _V + i1r
    o0, o1 = _threefry2x32(jnp.zeros((_N, _S), jnp.uint32),
                           flat.astype(jnp.uint32))
    bits = o0 ^ o1
    masked = (bits >> 31) == 0

    pad = s_iota >= len_ref[b]
    v1 = jnp.broadcast_to(m1, (_N, _S))
    v2 = jnp.broadcast_to(m2, (_N, _S))
    i2r = jnp.broadcast_to(i2, (_N, _S))

    sel_v = jnp.where(masked, v2, v1)
    sel_v = jnp.where(pad, 0.0, sel_v)
    a = jnp.sum(sel_v, axis=-1, keepdims=True)

    pred = jnp.where(masked, i2r, i1r)
    tgt = jnp.broadcast_to(tgt_ref[0, 0].reshape(1, _S), (_N, _S))
    err = jnp.sum(
        jnp.where(pad, 0.0, (tgt != pred).astype(jnp.float32)),
        axis=-1, keepdims=True)

    md = jnp.max(a, axis=0, keepdims=True)
    w = jnp.exp(a - md)
    normal = w / jnp.sum(w, axis=0, keepdims=True)
    dev = err - jnp.mean(err, axis=0, keepdims=True)
    loss_b = jnp.sum(normal * dev, axis=0, keepdims=True) * (1.0 / _B)

    @pl.when(b == 0)
    def _():
        out_ref[0] = jnp.zeros((1, 1), jnp.float32)

    out_ref[0] += loss_b


def kernel(logit, tgt, tgt_lens):
    xt = jnp.transpose(logit, (0, 2, 1))
    tgt3 = tgt.reshape(_B, 1, _S)
    loss = pl.pallas_call(
        _mwer_body,
        grid=(_B,),
        in_specs=[
            pl.BlockSpec(memory_space=pltpu.SMEM),
            pl.BlockSpec((1, _V, _S), lambda b: (b, 0, 0)),
            pl.BlockSpec((1, 1, _S), lambda b: (b, 0, 0)),
        ],
        out_specs=pl.BlockSpec((1, 1, 1), lambda b: (0, 0, 0)),
        out_shape=jax.ShapeDtypeStruct((1, 1, 1), jnp.float32),
    )(tgt_lens, xt, tgt3)
    return loss[0, 0, 0]
